# baseline (speedup 1.0000x reference)
import functools

import jax
import jax.numpy as jnp
from jax import lax
from jax.experimental import pallas as pl
from jax.experimental.pallas import tpu as pltpu
from jax.experimental.pallas import tpu_sc as plsc

E = 8
D = 1024
DFF = 4096
T = 2048
TK = 2 * T
TILE = 128
NTILES = 40
PMAX = NTILES * TILE
F_BLK = 512
NF = DFF // F_BLK


def _router_kernel(tokens_ref, rw_ref, p1_ref, p2_ref, dst1_ref, dst2_ref,
                   te_ref, tokbf_ref, oh1_s, oh2_s):
    tokbf_ref[...] = tokens_ref[...].astype(jnp.bfloat16)
    logits = jnp.dot(tokens_ref[...], rw_ref[...],
                     preferred_element_type=jnp.float32)
    idx = lax.broadcasted_iota(jnp.int32, (T, E), 1)
    m1 = jnp.max(logits, axis=1, keepdims=True)
    a1 = jnp.min(jnp.where(logits == m1, idx, E), axis=1, keepdims=True)
    masked = jnp.where(idx == a1, -jnp.inf, logits)
    m2 = jnp.max(masked, axis=1, keepdims=True)
    a2 = jnp.min(jnp.where(masked == m2, idx, E), axis=1, keepdims=True)
    p1 = 1.0 / (1.0 + jnp.exp(m2 - m1))
    p1_ref[...] = p1
    p2_ref[...] = 1.0 - p1

    oh1 = (idx == a1).astype(jnp.float32)
    oh2 = (idx == a2).astype(jnp.float32)
    oh1_s[...] = oh1
    oh2_s[...] = oh2

    r = lax.broadcasted_iota(jnp.int32, (TILE, TILE), 0)
    c = lax.broadcasted_iota(jnp.int32, (TILE, TILE), 1)
    ls = (c < r).astype(jnp.float32)

    def ranks(oh_ref, dref):
        def body(b, carry):
            rows = pl.ds(b * TILE, TILE)
            blk = oh_ref[rows, :]
            ex = jnp.dot(ls, blk, preferred_element_type=jnp.float32) + carry
            dref[rows, :] = jnp.sum(ex * blk, axis=1,
                                    keepdims=True).astype(jnp.int32)
            return carry + jnp.sum(blk, axis=0, keepdims=True)
        return lax.fori_loop(0, T // TILE, body, jnp.zeros((1, E),
                                                           jnp.float32))

    counts1 = ranks(oh1_s, dst1_ref)
    counts2 = ranks(oh2_s, dst2_ref)
    counts = counts1 + counts2

    nt = jnp.floor((counts + (TILE - 1.0)) * (1.0 / TILE))
    rr = lax.broadcasted_iota(jnp.int32, (E, E), 0)
    cc = lax.broadcasted_iota(jnp.int32, (E, E), 1)
    us = (rr < cc).astype(jnp.float32)
    ntb = jnp.broadcast_to(nt, (E, E))
    tile_off = jnp.dot(ntb, us, preferred_element_type=jnp.float32)[0:1, :]
    row_off = tile_off * float(TILE)

    off1 = jnp.sum(row_off * oh1, axis=1, keepdims=True)
    off2 = jnp.sum((row_off + counts1) * oh2, axis=1, keepdims=True)
    dst1_ref[...] = dst1_ref[...] + off1.astype(jnp.int32)
    dst2_ref[...] = dst2_ref[...] + off2.astype(jnp.int32)

    rows16 = lax.broadcasted_iota(jnp.int32, (2 * E, E), 0)
    lanes8 = lax.broadcasted_iota(jnp.int32, (2 * E, E), 1)
    pick_off = jnp.where((rows16 < E) & (lanes8 == rows16), 1.0, 0.0)
    pick_nt = jnp.where((rows16 >= E) & (lanes8 == rows16 - E), 1.0, 0.0)
    sinfo = (jnp.sum(jnp.broadcast_to(tile_off, (2 * E, E)) * pick_off
                     + jnp.broadcast_to(nt, (2 * E, E)) * pick_nt,
                     axis=1, keepdims=True))
    te_ref[...] = sinfo.astype(jnp.int32)


def _gemm_kernel(s_ref, x_ref, wg_ref, wu_ref, wd_ref, out_ref):
    e = pl.program_id(0)
    f = pl.program_id(1)
    start = s_ref[e]
    n = s_ref[E + e]

    wg = wg_ref[0].astype(jnp.bfloat16)
    wu = wu_ref[0].astype(jnp.bfloat16)
    wd = wd_ref[0].astype(jnp.bfloat16)

    def tile_body(q, carry):
        rows = pl.ds((start + q) * TILE, TILE)
        x = x_ref[rows, :]
        hg = jnp.dot(x, wg, preferred_element_type=jnp.float32)
        hu = jnp.dot(x, wu, preferred_element_type=jnp.float32)
        h = ((hg * jax.nn.sigmoid(hg)) * hu).astype(jnp.bfloat16)
        partial = jnp.dot(h, wd, preferred_element_type=jnp.float32)

        prev = jnp.where(f == 0, 0.0, out_ref[rows, :])
        out_ref[rows, :] = prev + partial
        return carry

    lax.fori_loop(0, n, tile_body, 0)


def _grouped_gemm(sinfo, x_pad, w_gate, w_up, w_down):
    spec = pltpu.PrefetchScalarGridSpec(
        num_scalar_prefetch=1,
        grid=(E, NF),
        in_specs=[
            pl.BlockSpec((PMAX, D), lambda e, f, s: (0, 0)),
            pl.BlockSpec((1, D, F_BLK), lambda e, f, s: (e, 0, f)),
            pl.BlockSpec((1, D, F_BLK), lambda e, f, s: (e, 0, f)),
            pl.BlockSpec((1, F_BLK, D), lambda e, f, s: (e, f, 0)),
        ],
        out_specs=pl.BlockSpec((PMAX, D), lambda e, f, s: (0, 0)),
    )
    return pl.pallas_call(
        _gemm_kernel,
        grid_spec=spec,
        out_shape=jax.ShapeDtypeStruct((PMAX, D), jnp.float32),
        compiler_params=pltpu.CompilerParams(
            vmem_limit_bytes=100 * 1024 * 1024),
    )(sinfo, x_pad, w_gate, w_up, w_down)


NC = 2
NS = 16


def _dispatch_sc(tokens, dst2d):
    nc = NC
    mesh = plsc.VectorSubcoreMesh(core_axis_name="c", subcore_axis_name="s")

    @functools.partial(
        pl.kernel,
        out_type=jax.ShapeDtypeStruct((PMAX, D // 2), jnp.int32),
        mesh=mesh,
        scratch_types=[
            pltpu.VMEM((2, 64), jnp.int32),
            pltpu.VMEM((64, D // 2), jnp.int32),
            pltpu.SemaphoreType.DMA,
        ],
    )
    def k(tokens_hbm, dst_hbm, xpad_hbm, idx_v, rows_v, sem):
        wid = lax.axis_index("s") * nc + lax.axis_index("c")
        pltpu.sync_copy(dst_hbm.at[pl.ds(wid * 2, 2)], idx_v)
        for cch in range(2):
            chunk = wid * 2 + cch
            t_base = lax.rem(chunk, 32) * 64
            pltpu.sync_copy(tokens_hbm.at[pl.ds(t_base, 64)], rows_v)
            pltpu.async_copy(rows_v, xpad_hbm.at[idx_v.at[cch]], sem).wait()

    return k(tokens, dst2d)


def _combine_sc(out_pad, dst1_2d, dst2_2d):
    nc = NC
    mesh = plsc.VectorSubcoreMesh(core_axis_name="c", subcore_axis_name="s")

    @functools.partial(
        pl.kernel,
        out_type=(jax.ShapeDtypeStruct((T, D), jnp.float32),
                  jax.ShapeDtypeStruct((T, D), jnp.float32)),
        mesh=mesh,
        scratch_types=[
            pltpu.VMEM((1, 64), jnp.int32),
            pltpu.VMEM((64, D), jnp.float32),
            pltpu.SemaphoreType.DMA,
        ],
    )
    def k(pad_hbm, d1_hbm, d2_hbm, g1_hbm, g2_hbm, idx_v, rows_v, sem):
        wid = lax.axis_index("s") * nc + lax.axis_index("c")
        base = wid * 64
        pltpu.sync_copy(d1_hbm.at[pl.ds(wid, 1)], idx_v)
        pltpu.async_copy(pad_hbm.at[idx_v.at[0]], rows_v, sem).wait()
        pltpu.sync_copy(rows_v, g1_hbm.at[pl.ds(base, 64)])
        pltpu.sync_copy(d2_hbm.at[pl.ds(wid, 1)], idx_v)
        pltpu.async_copy(pad_hbm.at[idx_v.at[0]], rows_v, sem).wait()
        pltpu.sync_copy(rows_v, g2_hbm.at[pl.ds(base, 64)])

    return k(out_pad, dst1_2d, dst2_2d)


def _final_kernel(p1_ref, p2_ref, g1_ref, g2_ref, out_ref):
    out_ref[...] = p1_ref[...] * g1_ref[...] + p2_ref[...] * g2_ref[...]


def kernel(x, router_w, w_gate, w_up, w_down):
    orig_shape = x.shape
    tokens = x.reshape(T, D)

    p1, p2, dst1, dst2, te_col, tok_bf = pl.pallas_call(
        _router_kernel,
        scratch_shapes=[pltpu.VMEM((T, E), jnp.float32),
                        pltpu.VMEM((T, E), jnp.float32)],
        out_shape=(
            jax.ShapeDtypeStruct((T, 1), jnp.float32),
            jax.ShapeDtypeStruct((T, 1), jnp.float32),
            jax.ShapeDtypeStruct((T, 1), jnp.int32),
            jax.ShapeDtypeStruct((T, 1), jnp.int32),
            jax.ShapeDtypeStruct((2 * E, 1), jnp.int32),
            jax.ShapeDtypeStruct((T, D), jnp.bfloat16),
        ),
    )(tokens, router_w)

    sinfo = te_col.reshape(2 * E)
    dst2d = jnp.concatenate([dst1, dst2], axis=0).reshape(64, 64)

    tok_i32 = lax.bitcast_convert_type(tok_bf.reshape(T, D // 2, 2),
                                       jnp.int32)
    x_pad_i32 = _dispatch_sc(tok_i32, dst2d)
    x_pad = lax.bitcast_convert_type(x_pad_i32,
                                     jnp.bfloat16).reshape(PMAX, D)
    out_pad = _grouped_gemm(sinfo, x_pad, w_gate, w_up, w_down)
    g1, g2 = _combine_sc(out_pad, dst1.reshape(32, 64), dst2.reshape(32, 64))

    out = pl.pallas_call(
        _final_kernel,
        out_shape=jax.ShapeDtypeStruct((T, D), jnp.float32),
    )(p1, p2, g1, g2)

    return out.reshape(orig_shape)


# speedup vs baseline: 1.8761x; 1.8761x over previous
import functools

import jax
import jax.numpy as jnp
from jax import lax
from jax.experimental import pallas as pl
from jax.experimental.pallas import tpu as pltpu
from jax.experimental.pallas import tpu_sc as plsc

E = 8
D = 1024
DFF = 4096
T = 2048
TK = 2 * T
TILE = 128
NTILES = 40
PMAX = NTILES * TILE
F_BLK = 512
NF = DFF // F_BLK


def _router_kernel(tokens_ref, rw_ref, p1_ref, p2_ref, dst1_ref, dst2_ref,
                   te_ref, oh1_s, oh2_s):
    logits = jnp.dot(tokens_ref[...], rw_ref[...],
                     preferred_element_type=jnp.float32)
    idx = lax.broadcasted_iota(jnp.int32, (T, E), 1)
    m1 = jnp.max(logits, axis=1, keepdims=True)
    a1 = jnp.min(jnp.where(logits == m1, idx, E), axis=1, keepdims=True)
    masked = jnp.where(idx == a1, -jnp.inf, logits)
    m2 = jnp.max(masked, axis=1, keepdims=True)
    a2 = jnp.min(jnp.where(masked == m2, idx, E), axis=1, keepdims=True)
    p1 = 1.0 / (1.0 + jnp.exp(m2 - m1))
    p1_ref[...] = p1
    p2_ref[...] = 1.0 - p1

    oh1 = (idx == a1).astype(jnp.float32)
    oh2 = (idx == a2).astype(jnp.float32)
    oh1_s[...] = oh1
    oh2_s[...] = oh2

    r = lax.broadcasted_iota(jnp.int32, (TILE, TILE), 0)
    c = lax.broadcasted_iota(jnp.int32, (TILE, TILE), 1)
    ls = (c < r).astype(jnp.float32)

    def ranks(oh_ref, dref):
        def body(b, carry):
            rows = pl.ds(b * TILE, TILE)
            blk = oh_ref[rows, :]
            ex = jnp.dot(ls, blk, preferred_element_type=jnp.float32) + carry
            dref[rows, :] = jnp.sum(ex * blk, axis=1,
                                    keepdims=True).astype(jnp.int32)
            return carry + jnp.sum(blk, axis=0, keepdims=True)
        return lax.fori_loop(0, T // TILE, body, jnp.zeros((1, E),
                                                           jnp.float32))

    counts1 = ranks(oh1_s, dst1_ref)
    counts2 = ranks(oh2_s, dst2_ref)
    counts = counts1 + counts2

    nt = jnp.floor((counts + (TILE - 1.0)) * (1.0 / TILE))
    rr = lax.broadcasted_iota(jnp.int32, (E, E), 0)
    cc = lax.broadcasted_iota(jnp.int32, (E, E), 1)
    us = (rr < cc).astype(jnp.float32)
    ntb = jnp.broadcast_to(nt, (E, E))
    tile_off = jnp.dot(ntb, us, preferred_element_type=jnp.float32)[0:1, :]
    row_off = tile_off * float(TILE)

    off1 = jnp.sum(row_off * oh1, axis=1, keepdims=True)
    off2 = jnp.sum((row_off + counts1) * oh2, axis=1, keepdims=True)
    dst1_ref[...] = dst1_ref[...] + off1.astype(jnp.int32)
    dst2_ref[...] = dst2_ref[...] + off2.astype(jnp.int32)

    rows16 = lax.broadcasted_iota(jnp.int32, (2 * E, E), 0)
    lanes8 = lax.broadcasted_iota(jnp.int32, (2 * E, E), 1)
    pick_off = jnp.where((rows16 < E) & (lanes8 == rows16), 1.0, 0.0)
    pick_nt = jnp.where((rows16 >= E) & (lanes8 == rows16 - E), 1.0, 0.0)
    sinfo = (jnp.sum(jnp.broadcast_to(tile_off, (2 * E, E)) * pick_off
                     + jnp.broadcast_to(nt, (2 * E, E)) * pick_nt,
                     axis=1, keepdims=True))
    te_ref[...] = sinfo.astype(jnp.int32)


def _gemm_kernel(s_ref, x_ref, wg_ref, wu_ref, wd_ref, out_ref):
    e = pl.program_id(0)
    f = pl.program_id(1)
    start = s_ref[e]
    n = s_ref[E + e]

    def ffn(rows):
        x = x_ref[rows, :]
        hg = jnp.dot(x, wg_ref[0], preferred_element_type=jnp.float32)
        hu = jnp.dot(x, wu_ref[0], preferred_element_type=jnp.float32)
        h = (hg * jax.nn.sigmoid(hg)) * hu
        partial = jnp.dot(h, wd_ref[0], preferred_element_type=jnp.float32)

        @pl.when(f == 0)
        def _():
            out_ref[rows, :] = partial

        @pl.when(f > 0)
        def _():
            out_ref[rows, :] = out_ref[rows, :] + partial

    n4 = n // 4

    def big_body(q, carry):
        ffn(pl.ds(start * TILE + q * (4 * TILE), 4 * TILE))
        return carry

    def small_body(q, carry):
        ffn(pl.ds((start + n4 * 4 + q) * TILE, TILE))
        return carry

    lax.fori_loop(0, n4, big_body, 0)
    lax.fori_loop(0, n - n4 * 4, small_body, 0)


def _grouped_gemm(sinfo, x_pad, w_gate, w_up, w_down):
    spec = pltpu.PrefetchScalarGridSpec(
        num_scalar_prefetch=1,
        grid=(E, NF),
        in_specs=[
            pl.BlockSpec((PMAX, D), lambda e, f, s: (0, 0)),
            pl.BlockSpec((1, D, F_BLK), lambda e, f, s: (e, 0, f)),
            pl.BlockSpec((1, D, F_BLK), lambda e, f, s: (e, 0, f)),
            pl.BlockSpec((1, F_BLK, D), lambda e, f, s: (e, f, 0)),
        ],
        out_specs=pl.BlockSpec((PMAX, D), lambda e, f, s: (0, 0)),
    )
    return pl.pallas_call(
        _gemm_kernel,
        grid_spec=spec,
        out_shape=jax.ShapeDtypeStruct((PMAX, D), jnp.float32),
        compiler_params=pltpu.CompilerParams(
            vmem_limit_bytes=100 * 1024 * 1024),
    )(sinfo, x_pad, w_gate, w_up, w_down)


NC = 2
NS = 16


def _dispatch_sc(tokens, dst2d):
    nc = NC
    mesh = plsc.VectorSubcoreMesh(core_axis_name="c", subcore_axis_name="s")

    @functools.partial(
        pl.kernel,
        out_type=jax.ShapeDtypeStruct((PMAX, D), jnp.float32),
        mesh=mesh,
        scratch_types=[
            pltpu.VMEM((2, 64), jnp.int32),
            pltpu.VMEM((64, D), jnp.float32),
            pltpu.SemaphoreType.DMA,
        ],
    )
    def k(tokens_hbm, dst_hbm, xpad_hbm, idx_v, rows_v, sem):
        wid = lax.axis_index("s") * nc + lax.axis_index("c")
        pltpu.sync_copy(dst_hbm.at[pl.ds(wid * 2, 2)], idx_v)
        for cch in range(2):
            chunk = wid * 2 + cch
            t_base = lax.rem(chunk, 32) * 64
            pltpu.sync_copy(tokens_hbm.at[pl.ds(t_base, 64)], rows_v)
            pltpu.async_copy(rows_v, xpad_hbm.at[idx_v.at[cch]], sem).wait()

    return k(tokens, dst2d)


def _combine_sc(out_pad, dst1_2d, dst2_2d):
    nc = NC
    mesh = plsc.VectorSubcoreMesh(core_axis_name="c", subcore_axis_name="s")

    @functools.partial(
        pl.kernel,
        out_type=(jax.ShapeDtypeStruct((T, D), jnp.float32),
                  jax.ShapeDtypeStruct((T, D), jnp.float32)),
        mesh=mesh,
        scratch_types=[
            pltpu.VMEM((1, 64), jnp.int32),
            pltpu.VMEM((64, D), jnp.float32),
            pltpu.SemaphoreType.DMA,
        ],
    )
    def k(pad_hbm, d1_hbm, d2_hbm, g1_hbm, g2_hbm, idx_v, rows_v, sem):
        wid = lax.axis_index("s") * nc + lax.axis_index("c")
        base = wid * 64
        pltpu.sync_copy(d1_hbm.at[pl.ds(wid, 1)], idx_v)
        pltpu.async_copy(pad_hbm.at[idx_v.at[0]], rows_v, sem).wait()
        pltpu.sync_copy(rows_v, g1_hbm.at[pl.ds(base, 64)])
        pltpu.sync_copy(d2_hbm.at[pl.ds(wid, 1)], idx_v)
        pltpu.async_copy(pad_hbm.at[idx_v.at[0]], rows_v, sem).wait()
        pltpu.sync_copy(rows_v, g2_hbm.at[pl.ds(base, 64)])

    return k(out_pad, dst1_2d, dst2_2d)


def _final_kernel(p1_ref, p2_ref, g1_ref, g2_ref, out_ref):
    out_ref[...] = p1_ref[...] * g1_ref[...] + p2_ref[...] * g2_ref[...]


def kernel(x, router_w, w_gate, w_up, w_down):
    orig_shape = x.shape
    tokens = x.reshape(T, D)

    p1, p2, dst1, dst2, te_col = pl.pallas_call(
        _router_kernel,
        scratch_shapes=[pltpu.VMEM((T, E), jnp.float32),
                        pltpu.VMEM((T, E), jnp.float32)],
        out_shape=(
            jax.ShapeDtypeStruct((T, 1), jnp.float32),
            jax.ShapeDtypeStruct((T, 1), jnp.float32),
            jax.ShapeDtypeStruct((T, 1), jnp.int32),
            jax.ShapeDtypeStruct((T, 1), jnp.int32),
            jax.ShapeDtypeStruct((2 * E, 1), jnp.int32),
        ),
    )(tokens, router_w)

    sinfo = te_col.reshape(2 * E)
    dst2d = jnp.concatenate([dst1, dst2], axis=0).reshape(64, 64)

    x_pad = _dispatch_sc(tokens, dst2d)
    out_pad = _grouped_gemm(sinfo, x_pad, w_gate, w_up, w_down)
    g1, g2 = _combine_sc(out_pad, dst1.reshape(32, 64), dst2.reshape(32, 64))

    out = pl.pallas_call(
        _final_kernel,
        out_shape=jax.ShapeDtypeStruct((T, D), jnp.float32),
    )(p1, p2, g1, g2)

    return out.reshape(orig_shape)


# speedup vs baseline: 1.9034x; 1.0146x over previous
import functools

import jax
import jax.numpy as jnp
from jax import lax
from jax.experimental import pallas as pl
from jax.experimental.pallas import tpu as pltpu
from jax.experimental.pallas import tpu_sc as plsc

E = 8
D = 1024
DFF = 4096
T = 2048
TK = 2 * T
TILE = 128
NTILES = 40
PMAX = NTILES * TILE
F_BLK = 512
NF = DFF // F_BLK


def _router_kernel(tokens_ref, rw_ref, p1_ref, p2_ref, dst1_ref, dst2_ref,
                   te_ref, oh1_s, oh2_s):
    logits = jnp.dot(tokens_ref[...], rw_ref[...],
                     preferred_element_type=jnp.float32)
    idx = lax.broadcasted_iota(jnp.int32, (T, E), 1)
    m1 = jnp.max(logits, axis=1, keepdims=True)
    a1 = jnp.min(jnp.where(logits == m1, idx, E), axis=1, keepdims=True)
    masked = jnp.where(idx == a1, -jnp.inf, logits)
    m2 = jnp.max(masked, axis=1, keepdims=True)
    a2 = jnp.min(jnp.where(masked == m2, idx, E), axis=1, keepdims=True)
    p1 = 1.0 / (1.0 + jnp.exp(m2 - m1))
    p1_ref[...] = p1
    p2_ref[...] = 1.0 - p1

    oh1 = (idx == a1).astype(jnp.float32)
    oh2 = (idx == a2).astype(jnp.float32)
    oh1_s[...] = oh1
    oh2_s[...] = oh2

    CB = 512
    r = lax.broadcasted_iota(jnp.int32, (CB, CB), 0)
    c = lax.broadcasted_iota(jnp.int32, (CB, CB), 1)
    ls = (c < r).astype(jnp.float32)

    def ranks(oh_ref, dref):
        def body(b, carry):
            rows = pl.ds(b * CB, CB)
            blk = oh_ref[rows, :]
            ex = jnp.dot(ls, blk, preferred_element_type=jnp.float32) + carry
            dref[rows, :] = jnp.sum(ex * blk, axis=1,
                                    keepdims=True).astype(jnp.int32)
            return carry + jnp.sum(blk, axis=0, keepdims=True)
        return lax.fori_loop(0, T // CB, body, jnp.zeros((1, E),
                                                         jnp.float32))

    counts1 = ranks(oh1_s, dst1_ref)
    counts2 = ranks(oh2_s, dst2_ref)
    counts = counts1 + counts2

    nt = jnp.floor((counts + (TILE - 1.0)) * (1.0 / TILE))
    rr = lax.broadcasted_iota(jnp.int32, (E, E), 0)
    cc = lax.broadcasted_iota(jnp.int32, (E, E), 1)
    us = (rr < cc).astype(jnp.float32)
    ntb = jnp.broadcast_to(nt, (E, E))
    tile_off = jnp.dot(ntb, us, preferred_element_type=jnp.float32)[0:1, :]
    row_off = tile_off * float(TILE)

    off1 = jnp.sum(row_off * oh1, axis=1, keepdims=True)
    off2 = jnp.sum((row_off + counts1) * oh2, axis=1, keepdims=True)
    dst1_ref[...] = dst1_ref[...] + off1.astype(jnp.int32)
    dst2_ref[...] = dst2_ref[...] + off2.astype(jnp.int32)

    rows16 = lax.broadcasted_iota(jnp.int32, (2 * E, E), 0)
    lanes8 = lax.broadcasted_iota(jnp.int32, (2 * E, E), 1)
    pick_off = jnp.where((rows16 < E) & (lanes8 == rows16), 1.0, 0.0)
    pick_nt = jnp.where((rows16 >= E) & (lanes8 == rows16 - E), 1.0, 0.0)
    sinfo = (jnp.sum(jnp.broadcast_to(tile_off, (2 * E, E)) * pick_off
                     + jnp.broadcast_to(nt, (2 * E, E)) * pick_nt,
                     axis=1, keepdims=True))
    te_ref[...] = sinfo.astype(jnp.int32)


def _gemm_kernel(s_ref, x_ref, wg_ref, wu_ref, wd_ref, out_ref):
    e = pl.program_id(0)
    f = pl.program_id(1)
    start = s_ref[e]
    n = s_ref[E + e]

    def ffn(rows):
        x = x_ref[rows, :]
        hg = jnp.dot(x, wg_ref[0], preferred_element_type=jnp.float32)
        hu = jnp.dot(x, wu_ref[0], preferred_element_type=jnp.float32)
        h = (hg * jax.nn.sigmoid(hg)) * hu
        partial = jnp.dot(h, wd_ref[0], preferred_element_type=jnp.float32)

        @pl.when(f == 0)
        def _():
            out_ref[rows, :] = partial

        @pl.when(f > 0)
        def _():
            out_ref[rows, :] = out_ref[rows, :] + partial

    n4 = n // 4

    def big_body(q, carry):
        ffn(pl.ds(start * TILE + q * (4 * TILE), 4 * TILE))
        return carry

    def small_body(q, carry):
        ffn(pl.ds((start + n4 * 4 + q) * TILE, TILE))
        return carry

    lax.fori_loop(0, n4, big_body, 0)
    lax.fori_loop(0, n - n4 * 4, small_body, 0)


def _grouped_gemm(sinfo, x_pad, w_gate, w_up, w_down):
    spec = pltpu.PrefetchScalarGridSpec(
        num_scalar_prefetch=1,
        grid=(E, NF),
        in_specs=[
            pl.BlockSpec((PMAX, D), lambda e, f, s: (0, 0)),
            pl.BlockSpec((1, D, F_BLK), lambda e, f, s: (e, 0, f)),
            pl.BlockSpec((1, D, F_BLK), lambda e, f, s: (e, 0, f)),
            pl.BlockSpec((1, F_BLK, D), lambda e, f, s: (e, f, 0)),
        ],
        out_specs=pl.BlockSpec((PMAX, D), lambda e, f, s: (0, 0)),
    )
    return pl.pallas_call(
        _gemm_kernel,
        grid_spec=spec,
        out_shape=jax.ShapeDtypeStruct((PMAX, D), jnp.float32),
        compiler_params=pltpu.CompilerParams(
            vmem_limit_bytes=100 * 1024 * 1024),
    )(sinfo, x_pad, w_gate, w_up, w_down)


NC = 2
NS = 16


def _dispatch_sc(tokens, dst2d):
    nc = NC
    mesh = plsc.VectorSubcoreMesh(core_axis_name="c", subcore_axis_name="s")

    @functools.partial(
        pl.kernel,
        out_type=jax.ShapeDtypeStruct((PMAX, D), jnp.float32),
        mesh=mesh,
        scratch_types=[
            pltpu.VMEM((2, 64), jnp.int32),
            pltpu.VMEM((64, D), jnp.float32),
            pltpu.SemaphoreType.DMA,
        ],
    )
    def k(tokens_hbm, dst_hbm, xpad_hbm, idx_v, rows_v, sem):
        wid = lax.axis_index("s") * nc + lax.axis_index("c")
        pltpu.sync_copy(dst_hbm.at[pl.ds(wid * 2, 2)], idx_v)
        for cch in range(2):
            chunk = wid * 2 + cch
            t_base = lax.rem(chunk, 32) * 64
            pltpu.sync_copy(tokens_hbm.at[pl.ds(t_base, 64)], rows_v)
            pltpu.async_copy(rows_v, xpad_hbm.at[idx_v.at[cch]], sem).wait()

    return k(tokens, dst2d)


def _combine_sc(out_pad, dst1_2d, dst2_2d):
    nc = NC
    mesh = plsc.VectorSubcoreMesh(core_axis_name="c", subcore_axis_name="s")

    @functools.partial(
        pl.kernel,
        out_type=(jax.ShapeDtypeStruct((T, D), jnp.float32),
                  jax.ShapeDtypeStruct((T, D), jnp.float32)),
        mesh=mesh,
        scratch_types=[
            pltpu.VMEM((1, 64), jnp.int32),
            pltpu.VMEM((64, D), jnp.float32),
            pltpu.SemaphoreType.DMA,
        ],
    )
    def k(pad_hbm, d1_hbm, d2_hbm, g1_hbm, g2_hbm, idx_v, rows_v, sem):
        wid = lax.axis_index("s") * nc + lax.axis_index("c")
        base = wid * 64
        pltpu.sync_copy(d1_hbm.at[pl.ds(wid, 1)], idx_v)
        pltpu.async_copy(pad_hbm.at[idx_v.at[0]], rows_v, sem).wait()
        pltpu.sync_copy(rows_v, g1_hbm.at[pl.ds(base, 64)])
        pltpu.sync_copy(d2_hbm.at[pl.ds(wid, 1)], idx_v)
        pltpu.async_copy(pad_hbm.at[idx_v.at[0]], rows_v, sem).wait()
        pltpu.sync_copy(rows_v, g2_hbm.at[pl.ds(base, 64)])

    return k(out_pad, dst1_2d, dst2_2d)


def _final_kernel(p1_ref, p2_ref, g1_ref, g2_ref, out_ref):
    out_ref[...] = p1_ref[...] * g1_ref[...] + p2_ref[...] * g2_ref[...]


def kernel(x, router_w, w_gate, w_up, w_down):
    orig_shape = x.shape
    tokens = x.reshape(T, D)

    p1, p2, dst1, dst2, te_col = pl.pallas_call(
        _router_kernel,
        scratch_shapes=[pltpu.VMEM((T, E), jnp.float32),
                        pltpu.VMEM((T, E), jnp.float32)],
        out_shape=(
            jax.ShapeDtypeStruct((T, 1), jnp.float32),
            jax.ShapeDtypeStruct((T, 1), jnp.float32),
            jax.ShapeDtypeStruct((T, 1), jnp.int32),
            jax.ShapeDtypeStruct((T, 1), jnp.int32),
            jax.ShapeDtypeStruct((2 * E, 1), jnp.int32),
        ),
    )(tokens, router_w)

    sinfo = te_col.reshape(2 * E)
    dst2d = jnp.concatenate([dst1, dst2], axis=0).reshape(64, 64)

    x_pad = _dispatch_sc(tokens, dst2d)
    out_pad = _grouped_gemm(sinfo, x_pad, w_gate, w_up, w_down)
    g1, g2 = _combine_sc(out_pad, dst1.reshape(32, 64), dst2.reshape(32, 64))

    out = pl.pallas_call(
        _final_kernel,
        out_shape=jax.ShapeDtypeStruct((T, D), jnp.float32),
    )(p1, p2, g1, g2)

    return out.reshape(orig_shape)
